# baseline (speedup 1.0000x reference)
import functools

import jax
import jax.numpy as jnp
from jax import lax
from jax.experimental import pallas as pl
from jax.experimental.pallas import tpu as pltpu

N_NODES = 10000
N_EDGES = 320000
N_GRAPHS = 64
EMB = 128
EMB_ID = 64
M_DIM = 64
MLP = 256
FOURIER = 4
POS = 3

CPAD = 16
TW = CPAD + EMB
PW = 80
NODE_BLK = 2000
EDGE_BLK = 2048
E_PAD = 327680

_call = pl.pallas_call


def _silu(x):
    return x * jax.nn.sigmoid(x)


def _init_body(aid_ref, idn_ref, coords_ref, ea_ref, ei_ref, wa_ref, wb_ref,
               b_ref, out_ref):
    aid = aid_ref[...]
    idn = idn_ref[...]
    B = aid.shape[0]
    oh_a = (aid == lax.broadcasted_iota(jnp.int32, (B, 11), 1)).astype(jnp.float32)
    oh_i = (idn == lax.broadcasted_iota(jnp.int32, (B, 2), 1)).astype(jnp.float32)
    feats = oh_a @ ea_ref[...]
    featid = oh_i @ ei_ref[...]
    h0 = _silu(featid @ wa_ref[...] + feats @ wb_ref[...] + b_ref[...])
    out_ref[:, :CPAD] = coords_ref[...]
    out_ref[:, CPAD:] = h0


def _init_table(atomids, identity, coords16, params):
    p = params
    wa = p["W_init"][:EMB_ID]
    wb = p["W_init"][EMB_ID:]
    b = p["b_init"].reshape(1, EMB)
    grid = N_NODES // NODE_BLK
    return _call(
        _init_body,
        grid=(grid,),
        in_specs=[
            pl.BlockSpec((NODE_BLK, 1), lambda i: (i, 0)),
            pl.BlockSpec((NODE_BLK, 1), lambda i: (i, 0)),
            pl.BlockSpec((NODE_BLK, CPAD), lambda i: (i, 0)),
            pl.BlockSpec((11, EMB), lambda i: (0, 0)),
            pl.BlockSpec((2, EMB_ID), lambda i: (0, 0)),
            pl.BlockSpec((EMB_ID, EMB), lambda i: (0, 0)),
            pl.BlockSpec((EMB, EMB), lambda i: (0, 0)),
            pl.BlockSpec((1, EMB), lambda i: (0, 0)),
        ],
        out_specs=pl.BlockSpec((NODE_BLK, TW), lambda i: (i, 0)),
        out_shape=jax.ShapeDtypeStruct((N_NODES, TW), jnp.float32),
    )(atomids.reshape(-1, 1), identity.reshape(-1, 1), coords16,
      p["emb_atom"], p["emb_id"], wa, wb, b)


def _edge_body(xs_ref, xd_ref, w1i_ref, w1j_ref, w1f_ref, b1_ref, w2_ref,
               b2_ref, wc1_ref, bc1_ref, wc2_ref, bc2_ref, out_ref):
    i = pl.program_id(0)
    xs = xs_ref[...]
    xd = xd_ref[...]
    B = xs.shape[0]
    rel = xs[:, :CPAD] - xd[:, :CPAD]
    d = jnp.sum(rel * rel, axis=1, keepdims=True)
    hs = jnp.concatenate([d * (0.5 ** k) for k in range(FOURIER)], axis=1)
    four = jnp.concatenate(
        [jnp.sin(hs), jnp.cos(hs), d,
         jnp.zeros((B, CPAD - 2 * FOURIER - 1), jnp.float32)], axis=1)
    x_j = xs[:, CPAD:]
    x_i = xd[:, CPAD:]
    h1 = _silu(x_i @ w1i_ref[...] + x_j @ w1j_ref[...] + four @ w1f_ref[...]
               + b1_ref[...])
    m = _silu(h1 @ w2_ref[...] + b2_ref[...])
    ch = _silu(m @ wc1_ref[...] + bc1_ref[...])
    cw = ch @ wc2_ref[...] + bc2_ref[...]
    wrel = cw * rel
    eid = i * B + lax.broadcasted_iota(jnp.int32, (B, 1), 0)
    valid = (eid < N_EDGES).astype(jnp.float32)
    payload = jnp.concatenate([m, wrel], axis=1) * valid
    li = lax.broadcasted_iota(jnp.int32, (B, PW), 1)
    payload = payload + jnp.where(li == (M_DIM + 3), valid, 0.0)
    out_ref[...] = payload


def _edge_mlp(xsrc, xdst, kp):
    w1i = kp["We1"][:EMB]
    w1j = kp["We1"][EMB:2 * EMB]
    w1f = jnp.pad(kp["We1"][2 * EMB:], ((0, CPAD - 9), (0, 0)))
    H = w1i.shape[1]
    grid = E_PAD // EDGE_BLK
    return _call(
        _edge_body,
        grid=(grid,),
        in_specs=[
            pl.BlockSpec((EDGE_BLK, TW), lambda i: (i, 0)),
            pl.BlockSpec((EDGE_BLK, TW), lambda i: (i, 0)),
            pl.BlockSpec((EMB, H), lambda i: (0, 0)),
            pl.BlockSpec((EMB, H), lambda i: (0, 0)),
            pl.BlockSpec((CPAD, H), lambda i: (0, 0)),
            pl.BlockSpec((1, H), lambda i: (0, 0)),
            pl.BlockSpec((H, M_DIM), lambda i: (0, 0)),
            pl.BlockSpec((1, M_DIM), lambda i: (0, 0)),
            pl.BlockSpec((M_DIM, 4 * M_DIM), lambda i: (0, 0)),
            pl.BlockSpec((1, 4 * M_DIM), lambda i: (0, 0)),
            pl.BlockSpec((4 * M_DIM, 1), lambda i: (0, 0)),
            pl.BlockSpec((1, 1), lambda i: (0, 0)),
        ],
        out_specs=pl.BlockSpec((EDGE_BLK, PW), lambda i: (i, 0)),
        out_shape=jax.ShapeDtypeStruct((E_PAD, PW), jnp.float32),
    )(xsrc, xdst, w1i, w1j, w1f, kp["be1"].reshape(1, H),
      kp["We2"], kp["be2"].reshape(1, M_DIM),
      kp["Wc1"], kp["bc1"].reshape(1, 4 * M_DIM),
      kp["Wc2"], kp["bc2"].reshape(1, 1))


def _node_body(tab_ref, agg_ref, wn1a_ref, wn1b_ref, bn1_ref, wn2_ref,
               bn2_ref, out_ref):
    tab = tab_ref[...]
    a = agg_ref[0] + agg_ref[1]
    B = a.shape[0]
    cnt = jnp.maximum(a[:, M_DIM + 3:M_DIM + 4], 1.0)
    m_i = a[:, :M_DIM] / cnt
    mh = a[:, M_DIM:] / cnt
    li = lax.broadcasted_iota(jnp.int32, (B, CPAD), 1)
    mh = jnp.where(li < POS, mh, 0.0)
    feats = tab[:, CPAD:]
    h = _silu(feats @ wn1a_ref[...] + m_i @ wn1b_ref[...] + bn1_ref[...])
    h = h @ wn2_ref[...] + bn2_ref[...]
    out_ref[:, :CPAD] = tab[:, :CPAD] + mh
    out_ref[:, CPAD:] = feats + h


def _node_update(table, agg2, kp):
    wn1a = kp["Wn1"][:EMB]
    wn1b = kp["Wn1"][EMB:]
    grid = N_NODES // NODE_BLK
    return _call(
        _node_body,
        grid=(grid,),
        in_specs=[
            pl.BlockSpec((NODE_BLK, TW), lambda i: (i, 0)),
            pl.BlockSpec((2, NODE_BLK, PW), lambda i: (0, i, 0)),
            pl.BlockSpec((EMB, 2 * EMB), lambda i: (0, 0)),
            pl.BlockSpec((M_DIM, 2 * EMB), lambda i: (0, 0)),
            pl.BlockSpec((1, 2 * EMB), lambda i: (0, 0)),
            pl.BlockSpec((2 * EMB, EMB), lambda i: (0, 0)),
            pl.BlockSpec((1, EMB), lambda i: (0, 0)),
        ],
        out_specs=pl.BlockSpec((NODE_BLK, TW), lambda i: (i, 0)),
        out_shape=jax.ShapeDtypeStruct((N_NODES, TW), jnp.float32),
    )(table, agg2, wn1a, wn1b, kp["bn1"].reshape(1, 2 * EMB),
      kp["Wn2"], kp["bn2"].reshape(1, EMB))


def _final_body(t0_ref, t1_ref, t2_ref, batch_ref,
                w1_ref, b1_ref, w2_ref, b2_ref, w3_ref, b3_ref,
                v1_ref, c1_ref, v2_ref, c2_ref, v3_ref, c3_ref,
                out_ref, acc_ref, cnt_ref):
    i = pl.program_id(0)
    n = pl.num_programs(0)

    @pl.when(i == 0)
    def _():
        acc_ref[...] = jnp.zeros_like(acc_ref)
        cnt_ref[...] = jnp.zeros_like(cnt_ref)

    f0 = t0_ref[...][:, CPAD:]
    f1 = t1_ref[...][:, CPAD:]
    f2 = t2_ref[...][:, CPAD:]
    h = _silu(jnp.concatenate([f0, f1, f2], axis=1))
    h = _silu(h @ w1_ref[...] + b1_ref[...])
    h = _silu(h @ w2_ref[...] + b2_ref[...])
    h = _silu(h @ w3_ref[...] + b3_ref[...])
    B = h.shape[0]
    oh = (batch_ref[...] ==
          lax.broadcasted_iota(jnp.int32, (B, N_GRAPHS), 1)).astype(jnp.float32)
    acc_ref[...] += lax.dot_general(oh, h, (((0,), (0,)), ((), ())))
    cnt_ref[...] += jnp.sum(oh, axis=0, keepdims=True)

    @pl.when(i == n - 1)
    def _():
        g = acc_ref[...] / jnp.maximum(cnt_ref[...].reshape(N_GRAPHS, 1), 1.0)
        g = _silu(g @ v1_ref[...] + c1_ref[...])
        g = _silu(g @ v2_ref[...] + c2_ref[...])
        out_ref[...] = g @ v3_ref[...] + c3_ref[...]


def _final_head(t0, t1, t2, batch, params):
    (w1, b1), (w2, b2), (w3, b3) = params["fnn"]
    (v1, c1), (v2, c2), (v3, c3) = params["fnn2"]
    grid = N_NODES // NODE_BLK
    tspec = pl.BlockSpec((NODE_BLK, TW), lambda i: (i, 0))
    return _call(
        _final_body,
        grid=(grid,),
        in_specs=[
            tspec, tspec, tspec,
            pl.BlockSpec((NODE_BLK, 1), lambda i: (i, 0)),
            pl.BlockSpec((3 * EMB, MLP), lambda i: (0, 0)),
            pl.BlockSpec((1, MLP), lambda i: (0, 0)),
            pl.BlockSpec((MLP, MLP), lambda i: (0, 0)),
            pl.BlockSpec((1, MLP), lambda i: (0, 0)),
            pl.BlockSpec((MLP, MLP), lambda i: (0, 0)),
            pl.BlockSpec((1, MLP), lambda i: (0, 0)),
            pl.BlockSpec((MLP, MLP), lambda i: (0, 0)),
            pl.BlockSpec((1, MLP), lambda i: (0, 0)),
            pl.BlockSpec((MLP, MLP), lambda i: (0, 0)),
            pl.BlockSpec((1, MLP), lambda i: (0, 0)),
            pl.BlockSpec((MLP, 1), lambda i: (0, 0)),
            pl.BlockSpec((1, 1), lambda i: (0, 0)),
        ],
        out_specs=pl.BlockSpec((N_GRAPHS, 1), lambda i: (0, 0)),
        out_shape=jax.ShapeDtypeStruct((N_GRAPHS, 1), jnp.float32),
        scratch_shapes=[pltpu.VMEM((N_GRAPHS, MLP), jnp.float32),
                        pltpu.VMEM((1, N_GRAPHS), jnp.float32)],
    )(t0, t1, t2, batch.reshape(-1, 1),
      w1, b1.reshape(1, -1), w2, b2.reshape(1, -1), w3, b3.reshape(1, -1),
      v1, c1.reshape(1, -1), v2, c2.reshape(1, -1), v3, c3.reshape(1, -1))


def _gather(table, src_pad, dst_pad):
    return table[src_pad], table[dst_pad]


def _scatter(payload, dst):
    agg = jax.ops.segment_sum(payload[:N_EDGES], dst, num_segments=N_NODES)
    return jnp.stack([agg, jnp.zeros_like(agg)])


def kernel(atomids, identity, coords, edge_index, batch, params):
    coords16 = jnp.pad(coords, ((0, 0), (0, CPAD - POS)))
    src = edge_index[0].astype(jnp.int32)
    dst = edge_index[1].astype(jnp.int32)
    src_pad = jnp.pad(src, (0, E_PAD - N_EDGES))
    dst_pad = jnp.pad(dst, (0, E_PAD - N_EDGES))

    table = _init_table(atomids.astype(jnp.int32), identity.astype(jnp.int32),
                        coords16, params)
    tables = [table]
    for kp in params["kernels"]:
        xsrc, xdst = _gather(tables[-1], src_pad, dst_pad)
        payload = _edge_mlp(xsrc, xdst, kp)
        agg2 = _scatter(payload, dst)
        tables.append(_node_update(tables[-1], agg2, kp))

    return _final_head(tables[0], tables[1], tables[2],
                       batch.astype(jnp.int32), params)


# speedup vs baseline: 1.7818x; 1.7818x over previous
import functools

import jax
import jax.numpy as jnp
from jax import lax
from jax.experimental import pallas as pl
from jax.experimental.pallas import tpu as pltpu

N_NODES = 10000
N_EDGES = 320000
N_GRAPHS = 64
EMB = 128
EMB_ID = 64
M_DIM = 64
MLP = 256
FOURIER = 4
POS = 3

CPAD = 16
PW = 128
NODE_BLK = 2000
EDGE_BLK = 2048
E_PAD = 327680

_call = pl.pallas_call


def _silu(x):
    return x * jax.nn.sigmoid(x)


def _init_body(aid_ref, idn_ref, ea_ref, ei_ref, wa_ref, wb_ref, b_ref,
               out_ref):
    aid = aid_ref[...]
    idn = idn_ref[...]
    B = aid.shape[0]
    oh_a = (aid == lax.broadcasted_iota(jnp.int32, (B, 11), 1)).astype(jnp.float32)
    oh_i = (idn == lax.broadcasted_iota(jnp.int32, (B, 2), 1)).astype(jnp.float32)
    feats = oh_a @ ea_ref[...]
    featid = oh_i @ ei_ref[...]
    out_ref[...] = _silu(featid @ wa_ref[...] + feats @ wb_ref[...] + b_ref[...])


def _init_feats(atomids, identity, params):
    p = params
    wa = p["W_init"][:EMB_ID]
    wb = p["W_init"][EMB_ID:]
    b = p["b_init"].reshape(1, EMB)
    grid = N_NODES // NODE_BLK
    return _call(
        _init_body,
        grid=(grid,),
        in_specs=[
            pl.BlockSpec((NODE_BLK, 1), lambda i: (i, 0)),
            pl.BlockSpec((NODE_BLK, 1), lambda i: (i, 0)),
            pl.BlockSpec((11, EMB), lambda i: (0, 0)),
            pl.BlockSpec((2, EMB_ID), lambda i: (0, 0)),
            pl.BlockSpec((EMB_ID, EMB), lambda i: (0, 0)),
            pl.BlockSpec((EMB, EMB), lambda i: (0, 0)),
            pl.BlockSpec((1, EMB), lambda i: (0, 0)),
        ],
        out_specs=pl.BlockSpec((NODE_BLK, EMB), lambda i: (i, 0)),
        out_shape=jax.ShapeDtypeStruct((N_NODES, EMB), jnp.float32),
    )(atomids.reshape(-1, 1), identity.reshape(-1, 1),
      p["emb_atom"], p["emb_id"], wa, wb, b)


def _edge_body(xj_ref, xi_ref, cs_ref, cd_ref, w1i_ref, w1j_ref, w1f_ref,
               b1_ref, w2_ref, b2_ref, wc1_ref, bc1_ref, wc2_ref, bc2_ref,
               out_ref):
    i = pl.program_id(0)
    x_j = xj_ref[...]
    x_i = xi_ref[...]
    B = x_j.shape[0]
    rel = cs_ref[...] - cd_ref[...]
    d = jnp.sum(rel * rel, axis=1, keepdims=True)
    hs = jnp.concatenate([d * (0.5 ** k) for k in range(FOURIER)], axis=1)
    four = jnp.concatenate(
        [jnp.sin(hs), jnp.cos(hs), d,
         jnp.zeros((B, CPAD - 2 * FOURIER - 1), jnp.float32)], axis=1)
    h1 = _silu(x_i @ w1i_ref[...] + x_j @ w1j_ref[...] + four @ w1f_ref[...]
               + b1_ref[...])
    m = _silu(h1 @ w2_ref[...] + b2_ref[...])
    ch = _silu(m @ wc1_ref[...] + bc1_ref[...])
    cw = ch @ wc2_ref[...] + bc2_ref[...]
    wrel = cw * rel
    eid = i * B + lax.broadcasted_iota(jnp.int32, (B, 1), 0)
    valid = (eid < N_EDGES).astype(jnp.float32)
    payload = jnp.concatenate(
        [m, wrel, jnp.zeros((B, PW - M_DIM - CPAD), jnp.float32)],
        axis=1) * valid
    li = lax.broadcasted_iota(jnp.int32, (B, PW), 1)
    out_ref[...] = payload + jnp.where(li == (M_DIM + 3), valid, 0.0)


def _edge_mlp(xj, xi, csrc, cdst, kp):
    w1i = kp["We1"][:EMB]
    w1j = kp["We1"][EMB:2 * EMB]
    w1f = jnp.pad(kp["We1"][2 * EMB:], ((0, CPAD - 9), (0, 0)))
    H = w1i.shape[1]
    grid = E_PAD // EDGE_BLK
    return _call(
        _edge_body,
        grid=(grid,),
        in_specs=[
            pl.BlockSpec((EDGE_BLK, EMB), lambda i: (i, 0)),
            pl.BlockSpec((EDGE_BLK, EMB), lambda i: (i, 0)),
            pl.BlockSpec((EDGE_BLK, CPAD), lambda i: (i, 0)),
            pl.BlockSpec((EDGE_BLK, CPAD), lambda i: (i, 0)),
            pl.BlockSpec((EMB, H), lambda i: (0, 0)),
            pl.BlockSpec((EMB, H), lambda i: (0, 0)),
            pl.BlockSpec((CPAD, H), lambda i: (0, 0)),
            pl.BlockSpec((1, H), lambda i: (0, 0)),
            pl.BlockSpec((H, M_DIM), lambda i: (0, 0)),
            pl.BlockSpec((1, M_DIM), lambda i: (0, 0)),
            pl.BlockSpec((M_DIM, 4 * M_DIM), lambda i: (0, 0)),
            pl.BlockSpec((1, 4 * M_DIM), lambda i: (0, 0)),
            pl.BlockSpec((4 * M_DIM, 1), lambda i: (0, 0)),
            pl.BlockSpec((1, 1), lambda i: (0, 0)),
        ],
        out_specs=pl.BlockSpec((EDGE_BLK, PW), lambda i: (i, 0)),
        out_shape=jax.ShapeDtypeStruct((E_PAD, PW), jnp.float32),
    )(xj, xi, csrc, cdst, w1i, w1j, w1f, kp["be1"].reshape(1, H),
      kp["We2"], kp["be2"].reshape(1, M_DIM),
      kp["Wc1"], kp["bc1"].reshape(1, 4 * M_DIM),
      kp["Wc2"], kp["bc2"].reshape(1, 1))


def _node_body(f_ref, c_ref, agg_ref, wn1a_ref, wn1b_ref, bn1_ref, wn2_ref,
               bn2_ref, fo_ref, co_ref):
    feats = f_ref[...]
    a = agg_ref[0] + agg_ref[1]
    B = a.shape[0]
    cnt = jnp.maximum(a[:, M_DIM + 3:M_DIM + 4], 1.0)
    m_i = a[:, :M_DIM] / cnt
    mh = a[:, M_DIM:M_DIM + CPAD] / cnt
    li = lax.broadcasted_iota(jnp.int32, (B, CPAD), 1)
    mh = jnp.where(li < POS, mh, 0.0)
    h = _silu(feats @ wn1a_ref[...] + m_i @ wn1b_ref[...] + bn1_ref[...])
    h = h @ wn2_ref[...] + bn2_ref[...]
    fo_ref[...] = feats + h
    co_ref[...] = c_ref[...] + mh


def _node_update(feats, coords16, agg2, kp):
    wn1a = kp["Wn1"][:EMB]
    wn1b = kp["Wn1"][EMB:]
    grid = N_NODES // NODE_BLK
    return _call(
        _node_body,
        grid=(grid,),
        in_specs=[
            pl.BlockSpec((NODE_BLK, EMB), lambda i: (i, 0)),
            pl.BlockSpec((NODE_BLK, CPAD), lambda i: (i, 0)),
            pl.BlockSpec((2, NODE_BLK, PW), lambda i: (0, i, 0)),
            pl.BlockSpec((EMB, 2 * EMB), lambda i: (0, 0)),
            pl.BlockSpec((M_DIM, 2 * EMB), lambda i: (0, 0)),
            pl.BlockSpec((1, 2 * EMB), lambda i: (0, 0)),
            pl.BlockSpec((2 * EMB, EMB), lambda i: (0, 0)),
            pl.BlockSpec((1, EMB), lambda i: (0, 0)),
        ],
        out_specs=[pl.BlockSpec((NODE_BLK, EMB), lambda i: (i, 0)),
                   pl.BlockSpec((NODE_BLK, CPAD), lambda i: (i, 0))],
        out_shape=[jax.ShapeDtypeStruct((N_NODES, EMB), jnp.float32),
                   jax.ShapeDtypeStruct((N_NODES, CPAD), jnp.float32)],
    )(feats, coords16, agg2, wn1a, wn1b, kp["bn1"].reshape(1, 2 * EMB),
      kp["Wn2"], kp["bn2"].reshape(1, EMB))


def _final_body(f0_ref, f1_ref, f2_ref, batch_ref,
                w1_ref, b1_ref, w2_ref, b2_ref, w3_ref, b3_ref,
                v1_ref, c1_ref, v2_ref, c2_ref, v3_ref, c3_ref,
                out_ref, acc_ref, cnt_ref):
    i = pl.program_id(0)
    n = pl.num_programs(0)

    @pl.when(i == 0)
    def _():
        acc_ref[...] = jnp.zeros_like(acc_ref)
        cnt_ref[...] = jnp.zeros_like(cnt_ref)

    h = _silu(jnp.concatenate([f0_ref[...], f1_ref[...], f2_ref[...]], axis=1))
    h = _silu(h @ w1_ref[...] + b1_ref[...])
    h = _silu(h @ w2_ref[...] + b2_ref[...])
    h = _silu(h @ w3_ref[...] + b3_ref[...])
    B = h.shape[0]
    oh = (batch_ref[...] ==
          lax.broadcasted_iota(jnp.int32, (B, N_GRAPHS), 1)).astype(jnp.float32)
    acc_ref[...] += lax.dot_general(oh, h, (((0,), (0,)), ((), ())))
    cnt_ref[...] += jnp.sum(oh, axis=0, keepdims=True)

    @pl.when(i == n - 1)
    def _():
        g = acc_ref[...] / jnp.maximum(cnt_ref[...].reshape(N_GRAPHS, 1), 1.0)
        g = _silu(g @ v1_ref[...] + c1_ref[...])
        g = _silu(g @ v2_ref[...] + c2_ref[...])
        out_ref[...] = g @ v3_ref[...] + c3_ref[...]


def _final_head(f0, f1, f2, batch, params):
    (w1, b1), (w2, b2), (w3, b3) = params["fnn"]
    (v1, c1), (v2, c2), (v3, c3) = params["fnn2"]
    grid = N_NODES // NODE_BLK
    fspec = pl.BlockSpec((NODE_BLK, EMB), lambda i: (i, 0))
    return _call(
        _final_body,
        grid=(grid,),
        in_specs=[
            fspec, fspec, fspec,
            pl.BlockSpec((NODE_BLK, 1), lambda i: (i, 0)),
            pl.BlockSpec((3 * EMB, MLP), lambda i: (0, 0)),
            pl.BlockSpec((1, MLP), lambda i: (0, 0)),
            pl.BlockSpec((MLP, MLP), lambda i: (0, 0)),
            pl.BlockSpec((1, MLP), lambda i: (0, 0)),
            pl.BlockSpec((MLP, MLP), lambda i: (0, 0)),
            pl.BlockSpec((1, MLP), lambda i: (0, 0)),
            pl.BlockSpec((MLP, MLP), lambda i: (0, 0)),
            pl.BlockSpec((1, MLP), lambda i: (0, 0)),
            pl.BlockSpec((MLP, MLP), lambda i: (0, 0)),
            pl.BlockSpec((1, MLP), lambda i: (0, 0)),
            pl.BlockSpec((MLP, 1), lambda i: (0, 0)),
            pl.BlockSpec((1, 1), lambda i: (0, 0)),
        ],
        out_specs=pl.BlockSpec((N_GRAPHS, 1), lambda i: (0, 0)),
        out_shape=jax.ShapeDtypeStruct((N_GRAPHS, 1), jnp.float32),
        scratch_shapes=[pltpu.VMEM((N_GRAPHS, MLP), jnp.float32),
                        pltpu.VMEM((1, N_GRAPHS), jnp.float32)],
    )(f0, f1, f2, batch.reshape(-1, 1),
      w1, b1.reshape(1, -1), w2, b2.reshape(1, -1), w3, b3.reshape(1, -1),
      v1, c1.reshape(1, -1), v2, c2.reshape(1, -1), v3, c3.reshape(1, -1))


NC, NS, LANES = 2, 16, 16
NW = NC * NS
CHUNK = 128
CPT = E_PAD // (NW * CHUNK)
EPW = E_PAD // NW
NSTRIPE = 624
NSTRIPE_LAST = N_NODES - NSTRIPE * (NS - 1)


def _sc_mesh():
    from jax.experimental.pallas import tpu_sc as plsc
    return plsc.VectorSubcoreMesh(core_axis_name="c", subcore_axis_name="s",
                                  num_cores=NC, num_subcores=NS)


def _gather_feats_body(tab_hbm, srcI_hbm, dstI_hbm, xj_hbm, xi_hbm,
                       sidx, didx, bufs, bufd, sem_s, sem_d):
    wid = lax.axis_index("s") * NC + lax.axis_index("c")
    pltpu.sync_copy(srcI_hbm.at[wid], sidx)
    pltpu.sync_copy(dstI_hbm.at[wid], didx)
    base = wid * EPW

    def body(j, carry):
        cs = pltpu.async_copy(tab_hbm.at[sidx.at[j]], bufs, sem_s)
        cd = pltpu.async_copy(tab_hbm.at[didx.at[j]], bufd, sem_d)
        cs.wait()
        cd.wait()
        row0 = base + j * CHUNK
        pltpu.sync_copy(bufs, xj_hbm.at[pl.ds(row0, CHUNK)])
        pltpu.sync_copy(bufd, xi_hbm.at[pl.ds(row0, CHUNK)])
        return carry

    lax.fori_loop(0, CPT, body, 0)


def _gather_feats(table, srcI, dstI):
    f = functools.partial(
        pl.kernel,
        out_type=[jax.ShapeDtypeStruct((E_PAD, EMB), jnp.float32),
                  jax.ShapeDtypeStruct((E_PAD, EMB), jnp.float32)],
        mesh=_sc_mesh(),
        scratch_types=[
            pltpu.VMEM((CPT, CHUNK), jnp.int32),
            pltpu.VMEM((CPT, CHUNK), jnp.int32),
            pltpu.VMEM((CHUNK, EMB), jnp.float32),
            pltpu.VMEM((CHUNK, EMB), jnp.float32),
            pltpu.SemaphoreType.DMA,
            pltpu.SemaphoreType.DMA,
        ],
    )(_gather_feats_body)
    return f(table, srcI, dstI)


def _gather_coords(coords16, srcI, dstI):
    f = functools.partial(
        pl.kernel,
        out_type=[jax.ShapeDtypeStruct((E_PAD, CPAD), jnp.float32),
                  jax.ShapeDtypeStruct((E_PAD, CPAD), jnp.float32)],
        mesh=_sc_mesh(),
        scratch_types=[
            pltpu.VMEM((CPT, CHUNK), jnp.int32),
            pltpu.VMEM((CPT, CHUNK), jnp.int32),
            pltpu.VMEM((CHUNK, CPAD), jnp.float32),
            pltpu.VMEM((CHUNK, CPAD), jnp.float32),
            pltpu.SemaphoreType.DMA,
            pltpu.SemaphoreType.DMA,
        ],
        compiler_params=pltpu.CompilerParams(use_tc_tiling_on_sc=False),
    )(_gather_feats_body)
    return f(coords16, srcI, dstI)


def _scatter_body(pay_hbm, dstI_hbm, zero_hbm, agg_hbm, didx, pbuf, shared):
    from jax.experimental.pallas import tpu_sc as plsc
    cid = lax.axis_index("c")
    sid = lax.axis_index("s")
    wid = sid * NC + cid
    pltpu.sync_copy(dstI_hbm.at[wid], didx)

    @pl.when(sid < NS - 1)
    def _():
        pltpu.sync_copy(zero_hbm.at[pl.ds(sid * NSTRIPE, NSTRIPE)],
                        shared.at[pl.ds(sid * NSTRIPE, NSTRIPE)])

    @pl.when(sid == NS - 1)
    def _():
        pltpu.sync_copy(zero_hbm.at[pl.ds(sid * NSTRIPE, NSTRIPE_LAST)],
                        shared.at[pl.ds(sid * NSTRIPE, NSTRIPE_LAST)])

    plsc.subcore_barrier()
    base = wid * EPW

    def body(j, carry):
        pltpu.sync_copy(pay_hbm.at[pl.ds(base + j * CHUNK, CHUNK)], pbuf)
        pltpu.sync_copy(pbuf, shared.at[didx.at[j]], add=True)
        return carry

    lax.fori_loop(0, CPT, body, 0)
    plsc.subcore_barrier()

    @pl.when(sid < NS - 1)
    def _():
        pltpu.sync_copy(shared.at[pl.ds(sid * NSTRIPE, NSTRIPE)],
                        agg_hbm.at[cid, pl.ds(sid * NSTRIPE, NSTRIPE)])

    @pl.when(sid == NS - 1)
    def _():
        pltpu.sync_copy(shared.at[pl.ds(sid * NSTRIPE, NSTRIPE_LAST)],
                        agg_hbm.at[cid, pl.ds(sid * NSTRIPE, NSTRIPE_LAST)])


def _scatter(payload, dstI, zeros_n):
    f = functools.partial(
        pl.kernel,
        out_type=jax.ShapeDtypeStruct((NC, N_NODES, PW), jnp.float32),
        mesh=_sc_mesh(),
        scratch_types=[
            pltpu.VMEM((CPT, CHUNK), jnp.int32),
            pltpu.VMEM((CHUNK, PW), jnp.float32),
            pltpu.VMEM_SHARED((N_NODES, PW), jnp.float32),
        ],
    )(_scatter_body)
    return f(payload, dstI, zeros_n)


def kernel(atomids, identity, coords, edge_index, batch, params):
    coords16 = jnp.pad(coords, ((0, 0), (0, CPAD - POS)))
    src = edge_index[0].astype(jnp.int32)
    dst = edge_index[1].astype(jnp.int32)
    srcI = jnp.pad(src, (0, E_PAD - N_EDGES)).reshape(NW, CPT, CHUNK)
    dstI = jnp.pad(dst, (0, E_PAD - N_EDGES)).reshape(NW, CPT, CHUNK)
    zeros_n = jnp.zeros((N_NODES, PW), jnp.float32)

    feats = _init_feats(atomids.astype(jnp.int32), identity.astype(jnp.int32),
                        params)
    flist = [feats]
    for kp in params["kernels"]:
        xj, xi = _gather_feats(flist[-1], srcI, dstI)
        csrc, cdst = _gather_coords(coords16, srcI, dstI)
        payload = _edge_mlp(xj, xi, csrc, cdst, kp)
        agg2 = _scatter(payload, dstI, zeros_n)
        fnew, coords16 = _node_update(flist[-1], coords16, agg2, kp)
        flist.append(fnew)

    return _final_head(flist[0], flist[1], flist[2],
                       batch.astype(jnp.int32), params)
